# Initial kernel scaffold; baseline (speedup 1.0000x reference)
import jax
import jax.numpy as jnp
from jax.experimental import pallas as pl


def kernel(x, edge_indices, weight, att, bias):
    raise NotImplementedError("write your pallas kernel here")


# speedup vs baseline: 10.4955x; 10.4955x over previous
import jax
import jax.numpy as jnp
from jax import lax
from jax.experimental import pallas as pl
from jax.experimental.pallas import tpu as pltpu
from jax.experimental.pallas import tpu_sc as plsc

N = 10000
E = 320000
C = 128
NEG_SLOPE = 0.2

NUM_SC = 2
NUM_TILES = 16
N_WORKERS = NUM_SC * NUM_TILES
EDGES_PER_WORKER = E // N_WORKERS
CHUNK = 80
N_CHUNKS = EDGES_PER_WORKER // CHUNK
NP = 10240
ROWS_PER_TILE = NP // NUM_TILES
ZROWS = 16

BLK = 1024
GRID = NP // BLK

_SC_MESH = plsc.VectorSubcoreMesh(
    core_axis_name="c", subcore_axis_name="s",
    num_cores=NUM_SC, num_subcores=NUM_TILES)
_SC_PARAMS = pltpu.CompilerParams(needs_layout_passes=False)


def _tc1_body(x_ref, w_ref, attc_ref, h_ref, a2_ref, md_ref, ms_ref):
    i = pl.program_id(0)
    h = jnp.dot(x_ref[...], w_ref[...], preferred_element_type=jnp.float32)
    h_ref[...] = h
    a2 = jnp.dot(h, attc_ref[...], preferred_element_type=jnp.float32)
    a2_ref[...] = a2
    bmd = jnp.max(a2[:, 0])
    bms = jnp.max(a2[:, 1])

    @pl.when(i == 0)
    def _():
        md_ref[...] = jnp.full((1, C), -3.4e38, jnp.float32)
        ms_ref[...] = jnp.full((1, C), -3.4e38, jnp.float32)

    md_ref[...] = jnp.maximum(md_ref[...], bmd)
    ms_ref[...] = jnp.maximum(ms_ref[...], bms)


def _tc1(x, weight, attc):
    return pl.pallas_call(
        _tc1_body,
        grid=(GRID,),
        in_specs=[
            pl.BlockSpec((BLK, C), lambda i: (i, 0)),
            pl.BlockSpec((C, C), lambda i: (0, 0)),
            pl.BlockSpec((C, 2), lambda i: (0, 0)),
        ],
        out_specs=[
            pl.BlockSpec((BLK, C), lambda i: (i, 0)),
            pl.BlockSpec((BLK, 2), lambda i: (i, 0)),
            pl.BlockSpec((1, C), lambda i: (0, 0)),
            pl.BlockSpec((1, C), lambda i: (0, 0)),
        ],
        out_shape=[
            jax.ShapeDtypeStruct((NP, C), jnp.float32),
            jax.ShapeDtypeStruct((NP, 2), jnp.float32),
            jax.ShapeDtypeStruct((1, C), jnp.float32),
            jax.ShapeDtypeStruct((1, C), jnp.float32),
        ],
    )(x, weight, attc)


def _sca_body(adst_hbm, asrc_hbm, src_hbm, dst_hbm, m_hbm,
              p_out, den_out,
              adst_v, asrc_v, src_v, dst_v, p_lin_v, den_v, mbuf_v,
              sem):
    cid = lax.axis_index("c")
    sid = lax.axis_index("s")
    wid = sid * NUM_SC + cid

    pltpu.sync_copy(adst_hbm, adst_v)
    pltpu.sync_copy(asrc_hbm, asrc_v)
    pltpu.sync_copy(m_hbm, mbuf_v)
    m_off = mbuf_v[pl.ds(0, 16)]

    zeros16 = jnp.zeros((16,), jnp.float32)

    def _zden(r, _):
        den_v[pl.ds(r * 16, 16)] = zeros16
        return 0
    lax.fori_loop(0, NP // 16, _zden, 0)

    ebase = wid * EDGES_PER_WORKER

    def _chunk(ci, _):
        off = ebase + ci * CHUNK
        pltpu.sync_copy(src_hbm.at[pl.ds(off, CHUNK)], src_v)
        pltpu.sync_copy(dst_hbm.at[pl.ds(off, CHUNK)], dst_v)

        for g in range(CHUNK // 16):
            s16 = src_v[pl.ds(g * 16, 16)]
            d16 = dst_v[pl.ds(g * 16, 16)]
            a_s = plsc.load_gather(asrc_v, [s16])
            a_d = plsc.load_gather(adst_v, [d16])
            al = a_s + a_d
            al = jnp.where(al >= 0.0, al, NEG_SLOPE * al) - m_off
            p = jnp.exp(al)
            p = jnp.where(s16 != d16, p, 0.0)
            p_lin_v[pl.ds(g * 16, 16)] = p

        def _pden(j, _):
            jv = jnp.full((16,), j, jnp.int32)
            pj = plsc.load_gather(p_lin_v, [jv])
            dj = plsc.load_gather(dst_v, [jv])
            cur = plsc.load_gather(den_v, [dj])
            plsc.store_scatter(den_v, [dj], cur + pj)
            return 0
        lax.fori_loop(0, CHUNK, _pden, 0)

        pltpu.sync_copy(p_lin_v, p_out.at[pl.ds(off, CHUNK)])
        return 0
    lax.fori_loop(0, N_CHUNKS, _chunk, 0)

    pltpu.sync_copy(den_v, den_out.at[wid])


def _sc_attention(adst, asrc, src, dst, m16):
    fn = pl.kernel(
        _sca_body,
        out_type=[
            jax.ShapeDtypeStruct((E,), jnp.float32),
            jax.ShapeDtypeStruct((N_WORKERS, NP), jnp.float32),
        ],
        mesh=_SC_MESH,
        compiler_params=_SC_PARAMS,
        scratch_types=[
            pltpu.VMEM((NP,), jnp.float32),
            pltpu.VMEM((NP,), jnp.float32),
            pltpu.VMEM((CHUNK,), jnp.int32),
            pltpu.VMEM((CHUNK,), jnp.int32),
            pltpu.VMEM((CHUNK,), jnp.float32),
            pltpu.VMEM((NP,), jnp.float32),
            pltpu.VMEM((16,), jnp.float32),
            pltpu.SemaphoreType.DMA,
        ],
    )
    return fn(adst, asrc, src, dst, m16)


def _scb_body(h_hbm, src_hbm, dst_hbm, p_hbm,
              acc_out,
              src_v, dst_v, p_lin_v, rows_v, zbuf,
              acc_s, sem):
    cid = lax.axis_index("c")
    sid = lax.axis_index("s")
    wid = sid * NUM_SC + cid

    zeros16 = jnp.zeros((16,), jnp.float32)

    def _zrow(r, _):
        for cc in range(8):
            zbuf[r, pl.ds(cc * 16, 16)] = zeros16
        return 0
    lax.fori_loop(0, ZROWS, _zrow, 0)

    rbase = sid * ROWS_PER_TILE

    def _zfill(k, _):
        pltpu.sync_copy(zbuf, acc_s.at[pl.ds(rbase + k * ZROWS, ZROWS)])
        return 0
    lax.fori_loop(0, ROWS_PER_TILE // ZROWS, _zfill, 0)

    plsc.subcore_barrier()

    ebase = wid * EDGES_PER_WORKER

    def _chunk(ci, _):
        off = ebase + ci * CHUNK
        pltpu.sync_copy(src_hbm.at[pl.ds(off, CHUNK)], src_v)
        pltpu.sync_copy(dst_hbm.at[pl.ds(off, CHUNK)], dst_v)
        pltpu.sync_copy(p_hbm.at[pl.ds(off, CHUNK)], p_lin_v)
        pltpu.async_copy(h_hbm.at[src_v], rows_v, sem).wait()

        def _mul(j, _):
            pj = plsc.load_gather(p_lin_v, [jnp.full((16,), j, jnp.int32)])
            for cc in range(8):
                sl = pl.ds(cc * 16, 16)
                rows_v[j, sl] = rows_v[j, sl] * pj
            return 0
        lax.fori_loop(0, CHUNK, _mul, 0)

        pltpu.sync_copy(rows_v, acc_s.at[dst_v], add=True)
        return 0
    lax.fori_loop(0, N_CHUNKS, _chunk, 0)

    plsc.subcore_barrier()

    def _cout(k, _):
        r0 = rbase + k * ZROWS
        pltpu.sync_copy(acc_s.at[pl.ds(r0, ZROWS)], acc_out.at[cid, pl.ds(r0, ZROWS)])
        return 0
    lax.fori_loop(0, ROWS_PER_TILE // ZROWS, _cout, 0)


def _sc_message(h, src, dst, p):
    fn = pl.kernel(
        _scb_body,
        out_type=jax.ShapeDtypeStruct((NUM_SC, NP, C), jnp.float32),
        mesh=_SC_MESH,
        compiler_params=_SC_PARAMS,
        scratch_types=[
            pltpu.VMEM((CHUNK,), jnp.int32),
            pltpu.VMEM((CHUNK,), jnp.int32),
            pltpu.VMEM((CHUNK,), jnp.float32),
            pltpu.VMEM((CHUNK, C), jnp.float32),
            pltpu.VMEM((ZROWS, C), jnp.float32),
            pltpu.VMEM_SHARED((NP, C), jnp.float32),
            pltpu.SemaphoreType.DMA,
        ],
    )
    return fn(h, src, dst, p)


def _tc2_body(acc_ref, den_ref, h_ref, a2_ref, m_ref, bias_ref, out_ref):
    i = pl.program_id(0)
    m_off = m_ref[0, 0]
    ad = a2_ref[pl.ds(i * BLK, BLK), 0:1]
    asrc = a2_ref[pl.ds(i * BLK, BLK), 1:2]
    al = ad + asrc
    al = jnp.where(al >= 0.0, al, NEG_SLOPE * al) - m_off
    p_self = jnp.exp(al)
    num = acc_ref[0] + acc_ref[1] + p_self * h_ref[...]
    den = jnp.sum(den_ref[:, pl.ds(i * BLK, BLK)], axis=0)[:, None] + p_self
    o = num / den + bias_ref[...]
    nrm = jnp.sqrt(jnp.sum(o * o, axis=1, keepdims=True))
    out_ref[...] = o / jnp.maximum(nrm, 1e-12)


def _tc2(acc, den, h, a2, m_sum, bias):
    return pl.pallas_call(
        _tc2_body,
        grid=(GRID,),
        in_specs=[
            pl.BlockSpec((NUM_SC, BLK, C), lambda i: (0, i, 0)),
            pl.BlockSpec((N_WORKERS, NP), lambda i: (0, 0)),
            pl.BlockSpec((BLK, C), lambda i: (i, 0)),
            pl.BlockSpec((NP, 2), lambda i: (0, 0)),
            pl.BlockSpec((1, C), lambda i: (0, 0)),
            pl.BlockSpec((1, C), lambda i: (0, 0)),
        ],
        out_specs=pl.BlockSpec((BLK, C), lambda i: (i, 0)),
        out_shape=jax.ShapeDtypeStruct((NP, C), jnp.float32),
    )(acc, den, h, a2, m_sum, bias)


@jax.jit
def kernel(x, edge_indices, weight, att, bias):
    attc = jnp.stack([att[0, 0, :C], att[0, 0, C:]], axis=1)
    xp = jnp.pad(x, ((0, NP - N), (0, 0)))
    h, a2, md, ms = _tc1(xp, weight, attc)
    m_sum = md + ms
    src = edge_indices[0].astype(jnp.int32)
    dst = edge_indices[1].astype(jnp.int32)
    adst = a2[:, 0]
    asrc = a2[:, 1]
    m16 = m_sum[0, :16]
    p, den = _sc_attention(adst, asrc, src, dst, m16)
    acc = _sc_message(h, src, dst, p)
    return _tc2(acc, den, h, a2, m_sum, bias.reshape(1, C))[:N]


# speedup vs baseline: 16.9562x; 1.6156x over previous
import jax
import jax.numpy as jnp
from jax import lax
from jax.experimental import pallas as pl
from jax.experimental.pallas import tpu as pltpu
from jax.experimental.pallas import tpu_sc as plsc

N = 10000
E = 320000
C = 128
NEG_SLOPE = 0.2

NUM_SC = 2
NUM_TILES = 16
N_WORKERS = NUM_SC * NUM_TILES
EDGES_PER_WORKER = E // N_WORKERS
CHUNK = 80
N_CHUNKS = EDGES_PER_WORKER // CHUNK
BATCH = 400
NP = 10240
ROWS_PER_TILE = NP // NUM_TILES
ZROWS = 16

BLK = 1024
GRID = NP // BLK

_SC_MESH = plsc.VectorSubcoreMesh(
    core_axis_name="c", subcore_axis_name="s",
    num_cores=NUM_SC, num_subcores=NUM_TILES)
_SC_PARAMS = pltpu.CompilerParams(needs_layout_passes=False)


def _tc1_body(x_ref, w_ref, attc_ref, h_ref, a2_ref, md_ref, ms_ref):
    i = pl.program_id(0)
    h = jnp.dot(x_ref[...], w_ref[...], preferred_element_type=jnp.float32)
    h_ref[...] = h
    a2 = jnp.dot(h, attc_ref[...], preferred_element_type=jnp.float32)
    a2_ref[...] = a2
    bmd = jnp.max(a2[:, 0])
    bms = jnp.max(a2[:, 1])

    @pl.when(i == 0)
    def _():
        md_ref[...] = jnp.full((1, C), -3.4e38, jnp.float32)
        ms_ref[...] = jnp.full((1, C), -3.4e38, jnp.float32)

    md_ref[...] = jnp.maximum(md_ref[...], bmd)
    ms_ref[...] = jnp.maximum(ms_ref[...], bms)


def _tc1(x, weight, attc):
    return pl.pallas_call(
        _tc1_body,
        grid=(GRID,),
        in_specs=[
            pl.BlockSpec((BLK, C), lambda i: (i, 0)),
            pl.BlockSpec((C, C), lambda i: (0, 0)),
            pl.BlockSpec((C, 2), lambda i: (0, 0)),
        ],
        out_specs=[
            pl.BlockSpec((BLK, C), lambda i: (i, 0)),
            pl.BlockSpec((BLK, 2), lambda i: (i, 0)),
            pl.BlockSpec((1, C), lambda i: (0, 0)),
            pl.BlockSpec((1, C), lambda i: (0, 0)),
        ],
        out_shape=[
            jax.ShapeDtypeStruct((NP, C), jnp.float32),
            jax.ShapeDtypeStruct((NP, 2), jnp.float32),
            jax.ShapeDtypeStruct((1, C), jnp.float32),
            jax.ShapeDtypeStruct((1, C), jnp.float32),
        ],
    )(x, weight, attc)


def _sca_body(adst_hbm, asrc_hbm, src_hbm, dst_hbm, m_hbm,
              p_out, den_out,
              adst_v, asrc_v, src_v, dst_v, p_all_v, den_v, mbuf_v,
              sem):
    cid = lax.axis_index("c")
    sid = lax.axis_index("s")
    wid = sid * NUM_SC + cid
    ebase = wid * EDGES_PER_WORKER

    pltpu.sync_copy(adst_hbm, adst_v)
    pltpu.sync_copy(asrc_hbm, asrc_v)
    pltpu.sync_copy(m_hbm, mbuf_v)
    pltpu.sync_copy(src_hbm.at[pl.ds(ebase, EDGES_PER_WORKER)], src_v)
    pltpu.sync_copy(dst_hbm.at[pl.ds(ebase, EDGES_PER_WORKER)], dst_v)
    m_off = mbuf_v[pl.ds(0, 16)]

    zeros16 = jnp.zeros((16,), jnp.float32)

    def _zden(r, _):
        den_v[pl.ds(r * 16, 16)] = zeros16
        return 0
    lax.fori_loop(0, NP // 16, _zden, 0)

    def _grp(g, _):
        s16 = src_v[pl.ds(g * 16, 16)]
        d16 = dst_v[pl.ds(g * 16, 16)]
        a_s = plsc.load_gather(asrc_v, [s16])
        a_d = plsc.load_gather(adst_v, [d16])
        al = a_s + a_d
        al = jnp.where(al >= 0.0, al, NEG_SLOPE * al) - m_off
        p = jnp.exp(al)
        p = jnp.where(s16 != d16, p, 0.0)
        p_all_v[pl.ds(g * 16, 16)] = p
        return 0
    lax.fori_loop(0, EDGES_PER_WORKER // 16, _grp, 0)

    def _pden(j, _):
        jv = jnp.full((16,), j, jnp.int32)
        pj = plsc.load_gather(p_all_v, [jv])
        dj = plsc.load_gather(dst_v, [jv])
        cur = plsc.load_gather(den_v, [dj])
        plsc.store_scatter(den_v, [dj], cur + pj)
        return 0
    lax.fori_loop(0, EDGES_PER_WORKER, _pden, 0)

    pltpu.sync_copy(p_all_v, p_out.at[pl.ds(ebase, EDGES_PER_WORKER)])
    pltpu.sync_copy(den_v, den_out.at[wid])


def _sc_attention(adst, asrc, src, dst, m16):
    fn = pl.kernel(
        _sca_body,
        out_type=[
            jax.ShapeDtypeStruct((E,), jnp.float32),
            jax.ShapeDtypeStruct((N_WORKERS, NP), jnp.float32),
        ],
        mesh=_SC_MESH,
        compiler_params=_SC_PARAMS,
        scratch_types=[
            pltpu.VMEM((NP,), jnp.float32),
            pltpu.VMEM((NP,), jnp.float32),
            pltpu.VMEM((EDGES_PER_WORKER,), jnp.int32),
            pltpu.VMEM((EDGES_PER_WORKER,), jnp.int32),
            pltpu.VMEM((EDGES_PER_WORKER,), jnp.float32),
            pltpu.VMEM((NP,), jnp.float32),
            pltpu.VMEM((16,), jnp.float32),
            pltpu.SemaphoreType.DMA,
        ],
    )
    return fn(adst, asrc, src, dst, m16)


def _scb_body(h_hbm, src_hbm, dst_hbm, p_hbm,
              acc_out,
              src_b, p_b, dst_a, dst_bb, rows_a, rows_b, zbuf,
              acc_s, sem_a, sem_b):
    cid = lax.axis_index("c")
    sid = lax.axis_index("s")
    wid = sid * NUM_SC + cid

    zeros16 = jnp.zeros((16,), jnp.float32)

    def _zrow(r, _):
        for cc in range(8):
            zbuf[r, pl.ds(cc * 16, 16)] = zeros16
        return 0
    lax.fori_loop(0, ZROWS, _zrow, 0)

    rbase = sid * ROWS_PER_TILE

    def _zfill(k, _):
        pltpu.sync_copy(zbuf, acc_s.at[pl.ds(rbase + k * ZROWS, ZROWS)])
        return 0
    lax.fori_loop(0, ROWS_PER_TILE // ZROWS, _zfill, 0)

    plsc.subcore_barrier()

    ebase = wid * EDGES_PER_WORKER
    bufs = ((dst_a, rows_a, sem_a), (dst_bb, rows_b, sem_b))

    def _mul(rows_v, pbase):
        def _one(j, _):
            pj = plsc.load_gather(p_b, [jnp.full((16,), pbase + j, jnp.int32)])
            for cc in range(8):
                sl = pl.ds(cc * 16, 16)
                rows_v[j, sl] = rows_v[j, sl] * pj
            return 0
        lax.fori_loop(0, CHUNK, _one, 0)

    def _batch(ob, _):
        obase = ebase + ob * BATCH
        pltpu.sync_copy(src_hbm.at[pl.ds(obase, BATCH)], src_b)
        pltpu.sync_copy(p_hbm.at[pl.ds(obase, BATCH)], p_b)
        pltpu.sync_copy(dst_hbm.at[pl.ds(obase, CHUNK)], bufs[0][0])
        gathers = [pltpu.async_copy(
            h_hbm.at[src_b.at[pl.ds(0, CHUNK)]], bufs[0][1], bufs[0][2])]
        for k in range(BATCH // CHUNK):
            dst_v, rows_v, sem = bufs[k % 2]
            if k + 1 < BATCH // CHUNK:
                dst_n, rows_n, sem_n = bufs[(k + 1) % 2]
                pltpu.sync_copy(
                    dst_hbm.at[pl.ds(obase + (k + 1) * CHUNK, CHUNK)], dst_n)
                gathers.append(pltpu.async_copy(
                    h_hbm.at[src_b.at[pl.ds((k + 1) * CHUNK, CHUNK)]],
                    rows_n, sem_n))
            gathers[k].wait()
            _mul(rows_v, k * CHUNK)
            pltpu.sync_copy(rows_v, acc_s.at[dst_v], add=True)
        return 0
    lax.fori_loop(0, EDGES_PER_WORKER // BATCH, _batch, 0)

    plsc.subcore_barrier()

    def _cout(k, _):
        r0 = rbase + k * ZROWS
        pltpu.sync_copy(acc_s.at[pl.ds(r0, ZROWS)], acc_out.at[cid, pl.ds(r0, ZROWS)])
        return 0
    lax.fori_loop(0, ROWS_PER_TILE // ZROWS, _cout, 0)


def _sc_message(h, src, dst, p):
    fn = pl.kernel(
        _scb_body,
        out_type=jax.ShapeDtypeStruct((NUM_SC, NP, C), jnp.float32),
        mesh=_SC_MESH,
        compiler_params=_SC_PARAMS,
        scratch_types=[
            pltpu.VMEM((BATCH,), jnp.int32),
            pltpu.VMEM((BATCH,), jnp.float32),
            pltpu.VMEM((CHUNK,), jnp.int32),
            pltpu.VMEM((CHUNK,), jnp.int32),
            pltpu.VMEM((CHUNK, C), jnp.float32),
            pltpu.VMEM((CHUNK, C), jnp.float32),
            pltpu.VMEM((ZROWS, C), jnp.float32),
            pltpu.VMEM_SHARED((NP, C), jnp.float32),
            pltpu.SemaphoreType.DMA,
            pltpu.SemaphoreType.DMA,
        ],
    )
    return fn(h, src, dst, p)


def _tc2_body(acc_ref, den_ref, h_ref, a2_ref, m_ref, bias_ref, out_ref):
    i = pl.program_id(0)
    m_off = m_ref[0, 0]
    ad = a2_ref[pl.ds(i * BLK, BLK), 0:1]
    asrc = a2_ref[pl.ds(i * BLK, BLK), 1:2]
    al = ad + asrc
    al = jnp.where(al >= 0.0, al, NEG_SLOPE * al) - m_off
    p_self = jnp.exp(al)
    num = acc_ref[0] + acc_ref[1] + p_self * h_ref[...]
    den = jnp.sum(den_ref[:, pl.ds(i * BLK, BLK)], axis=0)[:, None] + p_self
    o = num / den + bias_ref[...]
    nrm = jnp.sqrt(jnp.sum(o * o, axis=1, keepdims=True))
    out_ref[...] = o / jnp.maximum(nrm, 1e-12)


def _tc2(acc, den, h, a2, m_sum, bias):
    return pl.pallas_call(
        _tc2_body,
        grid=(GRID,),
        in_specs=[
            pl.BlockSpec((NUM_SC, BLK, C), lambda i: (0, i, 0)),
            pl.BlockSpec((N_WORKERS, NP), lambda i: (0, 0)),
            pl.BlockSpec((BLK, C), lambda i: (i, 0)),
            pl.BlockSpec((NP, 2), lambda i: (0, 0)),
            pl.BlockSpec((1, C), lambda i: (0, 0)),
            pl.BlockSpec((1, C), lambda i: (0, 0)),
        ],
        out_specs=pl.BlockSpec((BLK, C), lambda i: (i, 0)),
        out_shape=jax.ShapeDtypeStruct((NP, C), jnp.float32),
    )(acc, den, h, a2, m_sum, bias)


@jax.jit
def kernel(x, edge_indices, weight, att, bias):
    attc = jnp.stack([att[0, 0, :C], att[0, 0, C:]], axis=1)
    xp = jnp.pad(x, ((0, NP - N), (0, 0)))
    h, a2, md, ms = _tc1(xp, weight, attc)
    m_sum = md + ms
    src = edge_indices[0].astype(jnp.int32)
    dst = edge_indices[1].astype(jnp.int32)
    adst = a2[:, 0]
    asrc = a2[:, 1]
    m16 = m_sum[0, :16]
    p, den = _sc_attention(adst, asrc, src, dst, m16)
    acc = _sc_message(h, src, dst, p)
    return _tc2(acc, den, h, a2, m_sum, bias.reshape(1, C))[:N]


# speedup vs baseline: 20.0987x; 1.1853x over previous
import jax
import jax.numpy as jnp
from jax import lax
from jax.experimental import pallas as pl
from jax.experimental.pallas import tpu as pltpu
from jax.experimental.pallas import tpu_sc as plsc

N = 10000
E = 320000
C = 128
NEG_SLOPE = 0.2

NUM_SC = 2
NUM_TILES = 16
N_WORKERS = NUM_SC * NUM_TILES
EDGES_PER_WORKER = E // N_WORKERS
CHUNK = 80
N_CHUNKS = EDGES_PER_WORKER // CHUNK
BATCH = 2000
NP = 10240
ROWS_PER_TILE = NP // NUM_TILES
ZROWS = 16

BLK = 1024
GRID = NP // BLK

_SC_MESH = plsc.VectorSubcoreMesh(
    core_axis_name="c", subcore_axis_name="s",
    num_cores=NUM_SC, num_subcores=NUM_TILES)
_SC_PARAMS = pltpu.CompilerParams(needs_layout_passes=False)


def _tc1_body(x_ref, w_ref, attc_ref, h_ref, a2_ref, md_ref, ms_ref):
    i = pl.program_id(0)
    h = jnp.dot(x_ref[...], w_ref[...], preferred_element_type=jnp.float32)
    h_ref[...] = h
    a2 = jnp.dot(h, attc_ref[...], preferred_element_type=jnp.float32)
    a2_ref[...] = a2
    bmd = jnp.max(a2[:, 0])
    bms = jnp.max(a2[:, 1])

    @pl.when(i == 0)
    def _():
        md_ref[...] = jnp.full((1, C), -3.4e38, jnp.float32)
        ms_ref[...] = jnp.full((1, C), -3.4e38, jnp.float32)

    md_ref[...] = jnp.maximum(md_ref[...], bmd)
    ms_ref[...] = jnp.maximum(ms_ref[...], bms)


def _tc1(x, weight, attc):
    return pl.pallas_call(
        _tc1_body,
        grid=(GRID,),
        in_specs=[
            pl.BlockSpec((BLK, C), lambda i: (i, 0)),
            pl.BlockSpec((C, C), lambda i: (0, 0)),
            pl.BlockSpec((C, 2), lambda i: (0, 0)),
        ],
        out_specs=[
            pl.BlockSpec((BLK, C), lambda i: (i, 0)),
            pl.BlockSpec((BLK, 2), lambda i: (i, 0)),
            pl.BlockSpec((1, C), lambda i: (0, 0)),
            pl.BlockSpec((1, C), lambda i: (0, 0)),
        ],
        out_shape=[
            jax.ShapeDtypeStruct((NP, C), jnp.float32),
            jax.ShapeDtypeStruct((NP, 2), jnp.float32),
            jax.ShapeDtypeStruct((1, C), jnp.float32),
            jax.ShapeDtypeStruct((1, C), jnp.float32),
        ],
    )(x, weight, attc)


def _sca_body(adst_hbm, asrc_hbm, src_hbm, dst_hbm, m_hbm,
              p_out, den_out,
              adst_v, asrc_v, src_v, dst_v, p_all_v, den_v, mbuf_v,
              sem):
    cid = lax.axis_index("c")
    sid = lax.axis_index("s")
    wid = sid * NUM_SC + cid
    ebase = wid * EDGES_PER_WORKER

    pltpu.sync_copy(adst_hbm, adst_v)
    pltpu.sync_copy(asrc_hbm, asrc_v)
    pltpu.sync_copy(m_hbm, mbuf_v)
    pltpu.sync_copy(src_hbm.at[pl.ds(ebase, EDGES_PER_WORKER)], src_v)
    pltpu.sync_copy(dst_hbm.at[pl.ds(ebase, EDGES_PER_WORKER)], dst_v)
    m_off = mbuf_v[pl.ds(0, 16)]

    zeros16 = jnp.zeros((16,), jnp.float32)

    def _zden(r, _):
        den_v[pl.ds(r * 16, 16)] = zeros16
        return 0
    lax.fori_loop(0, NP // 16, _zden, 0)

    def _grp(g, _):
        s16 = src_v[pl.ds(g * 16, 16)]
        d16 = dst_v[pl.ds(g * 16, 16)]
        a_s = plsc.load_gather(asrc_v, [s16])
        a_d = plsc.load_gather(adst_v, [d16])
        al = a_s + a_d
        al = jnp.where(al >= 0.0, al, NEG_SLOPE * al) - m_off
        p = jnp.exp(al)
        p = jnp.where(s16 != d16, p, 0.0)
        p_all_v[pl.ds(g * 16, 16)] = p
        return 0
    lax.fori_loop(0, EDGES_PER_WORKER // 16, _grp, 0)

    def _pden(j, _):
        jv = jnp.full((16,), j, jnp.int32)
        pj = plsc.load_gather(p_all_v, [jv])
        dj = plsc.load_gather(dst_v, [jv])
        cur = plsc.load_gather(den_v, [dj])
        plsc.store_scatter(den_v, [dj], cur + pj)
        return 0
    lax.fori_loop(0, EDGES_PER_WORKER, _pden, 0)

    pltpu.sync_copy(p_all_v, p_out.at[pl.ds(ebase, EDGES_PER_WORKER)])
    pltpu.sync_copy(den_v, den_out.at[wid])


def _sc_attention(adst, asrc, src, dst, m16):
    fn = pl.kernel(
        _sca_body,
        out_type=[
            jax.ShapeDtypeStruct((E,), jnp.float32),
            jax.ShapeDtypeStruct((N_WORKERS, NP), jnp.float32),
        ],
        mesh=_SC_MESH,
        compiler_params=_SC_PARAMS,
        scratch_types=[
            pltpu.VMEM((NP,), jnp.float32),
            pltpu.VMEM((NP,), jnp.float32),
            pltpu.VMEM((EDGES_PER_WORKER,), jnp.int32),
            pltpu.VMEM((EDGES_PER_WORKER,), jnp.int32),
            pltpu.VMEM((EDGES_PER_WORKER,), jnp.float32),
            pltpu.VMEM((NP,), jnp.float32),
            pltpu.VMEM((16,), jnp.float32),
            pltpu.SemaphoreType.DMA,
        ],
    )
    return fn(adst, asrc, src, dst, m16)


def _scb_body(h_hbm, src_hbm, dst_hbm, p_hbm,
              acc_out,
              src_b, p_b, dst_a, dst_bb, rows_a, rows_b, zbuf,
              acc_s, sem_a, sem_b, ssem_a, ssem_b):
    cid = lax.axis_index("c")
    sid = lax.axis_index("s")
    wid = sid * NUM_SC + cid

    zeros16 = jnp.zeros((16,), jnp.float32)

    def _zrow(r, _):
        for cc in range(8):
            zbuf[r, pl.ds(cc * 16, 16)] = zeros16
        return 0
    lax.fori_loop(0, ZROWS, _zrow, 0)

    rbase = sid * ROWS_PER_TILE

    def _zfill(k, _):
        pltpu.sync_copy(zbuf, acc_s.at[pl.ds(rbase + k * ZROWS, ZROWS)])
        return 0
    lax.fori_loop(0, ROWS_PER_TILE // ZROWS, _zfill, 0)

    plsc.subcore_barrier()

    ebase = wid * EDGES_PER_WORKER
    bufs = ((dst_a, rows_a, sem_a, ssem_a), (dst_bb, rows_b, sem_b, ssem_b))
    n_chunks = BATCH // CHUNK

    def _mul(rows_v, pbase):
        @plsc.parallel_loop(0, CHUNK, unroll=2)
        def _one(j):
            pj = plsc.load_gather(p_b, [jnp.full((16,), pbase + j, jnp.int32)])
            for cc in range(8):
                sl = pl.ds(cc * 16, 16)
                rows_v[j, sl] = rows_v[j, sl] * pj

    def _batch(ob, _):
        obase = ebase + ob * BATCH
        pltpu.sync_copy(src_hbm.at[pl.ds(obase, BATCH)], src_b)
        pltpu.sync_copy(p_hbm.at[pl.ds(obase, BATCH)], p_b)
        pltpu.sync_copy(dst_hbm.at[pl.ds(obase, CHUNK)], bufs[0][0])
        gathers = [pltpu.async_copy(
            h_hbm.at[src_b.at[pl.ds(0, CHUNK)]], bufs[0][1], bufs[0][2])]
        scatters = [None, None]
        for k in range(n_chunks):
            dst_v, rows_v, sem, ssem = bufs[k % 2]
            if k + 1 < n_chunks:
                dst_n, rows_n, sem_n, _ = bufs[(k + 1) % 2]
                if scatters[(k + 1) % 2] is not None:
                    scatters[(k + 1) % 2].wait()
                    scatters[(k + 1) % 2] = None
                pltpu.sync_copy(
                    dst_hbm.at[pl.ds(obase + (k + 1) * CHUNK, CHUNK)], dst_n)
                gathers.append(pltpu.async_copy(
                    h_hbm.at[src_b.at[pl.ds((k + 1) * CHUNK, CHUNK)]],
                    rows_n, sem_n))
            gathers[k].wait()
            _mul(rows_v, k * CHUNK)
            scatters[k % 2] = pltpu.async_copy(
                rows_v, acc_s.at[dst_v], ssem, add=True)
        for s in scatters:
            if s is not None:
                s.wait()
        return 0
    lax.fori_loop(0, EDGES_PER_WORKER // BATCH, _batch, 0)

    plsc.subcore_barrier()

    def _cout(k, _):
        r0 = rbase + k * ZROWS
        pltpu.sync_copy(acc_s.at[pl.ds(r0, ZROWS)], acc_out.at[cid, pl.ds(r0, ZROWS)])
        return 0
    lax.fori_loop(0, ROWS_PER_TILE // ZROWS, _cout, 0)


def _sc_message(h, src, dst, p):
    fn = pl.kernel(
        _scb_body,
        out_type=jax.ShapeDtypeStruct((NUM_SC, NP, C), jnp.float32),
        mesh=_SC_MESH,
        compiler_params=_SC_PARAMS,
        scratch_types=[
            pltpu.VMEM((BATCH,), jnp.int32),
            pltpu.VMEM((BATCH,), jnp.float32),
            pltpu.VMEM((CHUNK,), jnp.int32),
            pltpu.VMEM((CHUNK,), jnp.int32),
            pltpu.VMEM((CHUNK, C), jnp.float32),
            pltpu.VMEM((CHUNK, C), jnp.float32),
            pltpu.VMEM((ZROWS, C), jnp.float32),
            pltpu.VMEM_SHARED((NP, C), jnp.float32),
            pltpu.SemaphoreType.DMA,
            pltpu.SemaphoreType.DMA,
            pltpu.SemaphoreType.DMA,
            pltpu.SemaphoreType.DMA,
        ],
    )
    return fn(h, src, dst, p)


def _tc2_body(acc_ref, den_ref, h_ref, a2_ref, m_ref, bias_ref, out_ref):
    i = pl.program_id(0)
    m_off = m_ref[0, 0]
    ad = a2_ref[pl.ds(i * BLK, BLK), 0:1]
    asrc = a2_ref[pl.ds(i * BLK, BLK), 1:2]
    al = ad + asrc
    al = jnp.where(al >= 0.0, al, NEG_SLOPE * al) - m_off
    p_self = jnp.exp(al)
    num = acc_ref[0] + acc_ref[1] + p_self * h_ref[...]
    den = jnp.sum(den_ref[:, pl.ds(i * BLK, BLK)], axis=0)[:, None] + p_self
    o = num / den + bias_ref[...]
    nrm = jnp.sqrt(jnp.sum(o * o, axis=1, keepdims=True))
    out_ref[...] = o / jnp.maximum(nrm, 1e-12)


def _tc2(acc, den, h, a2, m_sum, bias):
    return pl.pallas_call(
        _tc2_body,
        grid=(GRID,),
        in_specs=[
            pl.BlockSpec((NUM_SC, BLK, C), lambda i: (0, i, 0)),
            pl.BlockSpec((N_WORKERS, NP), lambda i: (0, 0)),
            pl.BlockSpec((BLK, C), lambda i: (i, 0)),
            pl.BlockSpec((NP, 2), lambda i: (0, 0)),
            pl.BlockSpec((1, C), lambda i: (0, 0)),
            pl.BlockSpec((1, C), lambda i: (0, 0)),
        ],
        out_specs=pl.BlockSpec((BLK, C), lambda i: (i, 0)),
        out_shape=jax.ShapeDtypeStruct((NP, C), jnp.float32),
    )(acc, den, h, a2, m_sum, bias)


@jax.jit
def kernel(x, edge_indices, weight, att, bias):
    attc = jnp.stack([att[0, 0, :C], att[0, 0, C:]], axis=1)
    xp = jnp.pad(x, ((0, NP - N), (0, 0)))
    h, a2, md, ms = _tc1(xp, weight, attc)
    m_sum = md + ms
    src = edge_indices[0].astype(jnp.int32)
    dst = edge_indices[1].astype(jnp.int32)
    adst = a2[:, 0]
    asrc = a2[:, 1]
    m16 = m_sum[0, :16]
    p, den = _sc_attention(adst, asrc, src, dst, m16)
    acc = _sc_message(h, src, dst, p)
    return _tc2(acc, den, h, a2, m_sum, bias.reshape(1, C))[:N]


# speedup vs baseline: 26.6141x; 1.3242x over previous
import jax
import jax.numpy as jnp
from jax import lax
from jax.experimental import pallas as pl
from jax.experimental.pallas import tpu as pltpu
from jax.experimental.pallas import tpu_sc as plsc

N = 10000
E = 320000
C = 128
NEG_SLOPE = 0.2

NUM_SC = 2
NUM_TILES = 16
N_WORKERS = NUM_SC * NUM_TILES
EDGES_PER_WORKER = E // N_WORKERS
CHUNK = 80
N_CHUNKS = EDGES_PER_WORKER // CHUNK
BATCH = 2000
NP = 10240
ROWS_PER_TILE = NP // NUM_TILES
ZROWS = 16

BLK = 1024
GRID = NP // BLK

_SC_MESH = plsc.VectorSubcoreMesh(
    core_axis_name="c", subcore_axis_name="s",
    num_cores=NUM_SC, num_subcores=NUM_TILES)
_SC_PARAMS = pltpu.CompilerParams(needs_layout_passes=False)


def _tc1_body(x_ref, w_ref, attc_ref, h_ref, a2_ref, md_ref, ms_ref):
    i = pl.program_id(0)
    h = jnp.dot(x_ref[...], w_ref[...], preferred_element_type=jnp.float32)
    h_ref[...] = h
    a2 = jnp.dot(h, attc_ref[...], preferred_element_type=jnp.float32)
    a2_ref[...] = a2
    bmd = jnp.max(a2[:, 0])
    bms = jnp.max(a2[:, 1])

    @pl.when(i == 0)
    def _():
        md_ref[...] = jnp.full((1, C), -3.4e38, jnp.float32)
        ms_ref[...] = jnp.full((1, C), -3.4e38, jnp.float32)

    md_ref[...] = jnp.maximum(md_ref[...], bmd)
    ms_ref[...] = jnp.maximum(ms_ref[...], bms)


def _tc1(x, weight, attc):
    return pl.pallas_call(
        _tc1_body,
        grid=(GRID,),
        in_specs=[
            pl.BlockSpec((BLK, C), lambda i: (i, 0)),
            pl.BlockSpec((C, C), lambda i: (0, 0)),
            pl.BlockSpec((C, 2), lambda i: (0, 0)),
        ],
        out_specs=[
            pl.BlockSpec((BLK, C), lambda i: (i, 0)),
            pl.BlockSpec((BLK, 2), lambda i: (i, 0)),
            pl.BlockSpec((1, C), lambda i: (0, 0)),
            pl.BlockSpec((1, C), lambda i: (0, 0)),
        ],
        out_shape=[
            jax.ShapeDtypeStruct((NP, C), jnp.float32),
            jax.ShapeDtypeStruct((NP, 2), jnp.float32),
            jax.ShapeDtypeStruct((1, C), jnp.float32),
            jax.ShapeDtypeStruct((1, C), jnp.float32),
        ],
    )(x, weight, attc)


def _sca_body(adst_hbm, asrc_hbm, src_hbm, dst_hbm, m_hbm,
              p_out, den_out,
              adst_v, asrc_v, src_v, dst_v, p_all_v, den_v, den2_v, den3_v,
              den4_v, mbuf_v, sem):
    cid = lax.axis_index("c")
    sid = lax.axis_index("s")
    wid = sid * NUM_SC + cid
    ebase = wid * EDGES_PER_WORKER

    cps = [
        pltpu.async_copy(adst_hbm, adst_v, sem),
        pltpu.async_copy(asrc_hbm, asrc_v, sem),
        pltpu.async_copy(m_hbm, mbuf_v, sem),
        pltpu.async_copy(src_hbm.at[pl.ds(ebase, EDGES_PER_WORKER)], src_v, sem),
        pltpu.async_copy(dst_hbm.at[pl.ds(ebase, EDGES_PER_WORKER)], dst_v, sem),
    ]

    zeros16 = jnp.zeros((16,), jnp.float32)
    dens = (den_v, den2_v, den3_v, den4_v)

    def _zden(r, _):
        for dv in dens:
            dv[pl.ds(r * 16, 16)] = zeros16
        return 0
    lax.fori_loop(0, NP // 16, _zden, 0)

    for cp in cps:
        cp.wait()
    m_off = mbuf_v[pl.ds(0, 16)]

    @plsc.parallel_loop(0, EDGES_PER_WORKER // 16, unroll=2)
    def _grp(g):
        s16 = src_v[pl.ds(g * 16, 16)]
        d16 = dst_v[pl.ds(g * 16, 16)]
        a_s = plsc.load_gather(asrc_v, [s16])
        a_d = plsc.load_gather(adst_v, [d16])
        al = a_s + a_d
        al = jnp.where(al >= 0.0, al, NEG_SLOPE * al) - m_off
        p = jnp.exp(al)
        p = jnp.where(s16 != d16, p, 0.0)
        p_all_v[pl.ds(g * 16, 16)] = p

    pw = pltpu.async_copy(
        p_all_v, p_out.at[pl.ds(ebase, EDGES_PER_WORKER)], sem)

    def _pden(q, _):
        j0 = q * 4
        for t in range(4):
            jv = jnp.full((16,), j0 + t, jnp.int32)
            pj = plsc.load_gather(p_all_v, [jv])
            dj = plsc.load_gather(dst_v, [jv])
            cur = plsc.load_gather(dens[t], [dj])
            plsc.store_scatter(dens[t], [dj], cur + pj)
        return 0
    lax.fori_loop(0, EDGES_PER_WORKER // 4, _pden, 0)

    def _dmerge(r, _):
        sl = pl.ds(r * 16, 16)
        den_v[sl] = (den_v[sl] + den2_v[sl]) + (den3_v[sl] + den4_v[sl])
        return 0
    lax.fori_loop(0, NP // 16, _dmerge, 0)

    pw.wait()
    pltpu.sync_copy(den_v, den_out.at[wid])


def _sc_attention(adst, asrc, src, dst, m16):
    fn = pl.kernel(
        _sca_body,
        out_type=[
            jax.ShapeDtypeStruct((E,), jnp.float32),
            jax.ShapeDtypeStruct((N_WORKERS, NP), jnp.float32),
        ],
        mesh=_SC_MESH,
        compiler_params=_SC_PARAMS,
        scratch_types=[
            pltpu.VMEM((NP,), jnp.float32),
            pltpu.VMEM((NP,), jnp.float32),
            pltpu.VMEM((EDGES_PER_WORKER,), jnp.int32),
            pltpu.VMEM((EDGES_PER_WORKER,), jnp.int32),
            pltpu.VMEM((EDGES_PER_WORKER,), jnp.float32),
            pltpu.VMEM((NP,), jnp.float32),
            pltpu.VMEM((NP,), jnp.float32),
            pltpu.VMEM((NP,), jnp.float32),
            pltpu.VMEM((NP,), jnp.float32),
            pltpu.VMEM((16,), jnp.float32),
            pltpu.SemaphoreType.DMA,
        ],
    )
    return fn(adst, asrc, src, dst, m16)


def _scb_body(h_hbm, src_hbm, dst_hbm, p_hbm,
              acc_out,
              src_b, p_b, dst_a, dst_bb, rows_a, rows_b, zbuf,
              acc_s, sem_a, sem_b, ssem_a, ssem_b):
    cid = lax.axis_index("c")
    sid = lax.axis_index("s")
    wid = sid * NUM_SC + cid

    zeros16 = jnp.zeros((16,), jnp.float32)

    def _zrow(r, _):
        for cc in range(8):
            zbuf[r, pl.ds(cc * 16, 16)] = zeros16
        return 0
    lax.fori_loop(0, ZROWS, _zrow, 0)

    rbase = sid * ROWS_PER_TILE

    def _zfill(k, _):
        pltpu.sync_copy(zbuf, acc_s.at[pl.ds(rbase + k * ZROWS, ZROWS)])
        return 0
    lax.fori_loop(0, ROWS_PER_TILE // ZROWS, _zfill, 0)

    plsc.subcore_barrier()

    ebase = wid * EDGES_PER_WORKER
    bufs = ((dst_a, rows_a, sem_a, ssem_a), (dst_bb, rows_b, sem_b, ssem_b))
    n_chunks = BATCH // CHUNK

    def _mul(rows_v, pbase):
        @plsc.parallel_loop(0, CHUNK, unroll=2)
        def _one(j):
            pj = plsc.load_gather(p_b, [jnp.full((16,), pbase + j, jnp.int32)])
            for cc in range(8):
                sl = pl.ds(cc * 16, 16)
                rows_v[j, sl] = rows_v[j, sl] * pj

    def _batch(ob, _):
        obase = ebase + ob * BATCH
        pltpu.sync_copy(src_hbm.at[pl.ds(obase, BATCH)], src_b)
        pltpu.sync_copy(p_hbm.at[pl.ds(obase, BATCH)], p_b)
        pltpu.sync_copy(dst_hbm.at[pl.ds(obase, CHUNK)], bufs[0][0])
        gathers = [pltpu.async_copy(
            h_hbm.at[src_b.at[pl.ds(0, CHUNK)]], bufs[0][1], bufs[0][2])]
        scatters = [None, None]
        for k in range(n_chunks):
            dst_v, rows_v, sem, ssem = bufs[k % 2]
            if k + 1 < n_chunks:
                dst_n, rows_n, sem_n, _ = bufs[(k + 1) % 2]
                if scatters[(k + 1) % 2] is not None:
                    scatters[(k + 1) % 2].wait()
                    scatters[(k + 1) % 2] = None
                pltpu.sync_copy(
                    dst_hbm.at[pl.ds(obase + (k + 1) * CHUNK, CHUNK)], dst_n)
                gathers.append(pltpu.async_copy(
                    h_hbm.at[src_b.at[pl.ds((k + 1) * CHUNK, CHUNK)]],
                    rows_n, sem_n))
            gathers[k].wait()
            _mul(rows_v, k * CHUNK)
            scatters[k % 2] = pltpu.async_copy(
                rows_v, acc_s.at[dst_v], ssem, add=True)
        for s in scatters:
            if s is not None:
                s.wait()
        return 0
    lax.fori_loop(0, EDGES_PER_WORKER // BATCH, _batch, 0)

    plsc.subcore_barrier()

    def _cout(k, _):
        r0 = rbase + k * ZROWS
        pltpu.sync_copy(acc_s.at[pl.ds(r0, ZROWS)], acc_out.at[cid, pl.ds(r0, ZROWS)])
        return 0
    lax.fori_loop(0, ROWS_PER_TILE // ZROWS, _cout, 0)


def _sc_message(h, src, dst, p):
    fn = pl.kernel(
        _scb_body,
        out_type=jax.ShapeDtypeStruct((NUM_SC, NP, C), jnp.float32),
        mesh=_SC_MESH,
        compiler_params=_SC_PARAMS,
        scratch_types=[
            pltpu.VMEM((BATCH,), jnp.int32),
            pltpu.VMEM((BATCH,), jnp.float32),
            pltpu.VMEM((CHUNK,), jnp.int32),
            pltpu.VMEM((CHUNK,), jnp.int32),
            pltpu.VMEM((CHUNK, C), jnp.float32),
            pltpu.VMEM((CHUNK, C), jnp.float32),
            pltpu.VMEM((ZROWS, C), jnp.float32),
            pltpu.VMEM_SHARED((NP, C), jnp.float32),
            pltpu.SemaphoreType.DMA,
            pltpu.SemaphoreType.DMA,
            pltpu.SemaphoreType.DMA,
            pltpu.SemaphoreType.DMA,
        ],
    )
    return fn(h, src, dst, p)


def _tc2_body(acc_ref, den_ref, h_ref, a2_ref, m_ref, bias_ref, out_ref):
    i = pl.program_id(0)
    m_off = m_ref[0, 0]
    ad = a2_ref[pl.ds(i * BLK, BLK), 0:1]
    asrc = a2_ref[pl.ds(i * BLK, BLK), 1:2]
    al = ad + asrc
    al = jnp.where(al >= 0.0, al, NEG_SLOPE * al) - m_off
    p_self = jnp.exp(al)
    num = acc_ref[0] + acc_ref[1] + p_self * h_ref[...]
    den = jnp.sum(den_ref[:, pl.ds(i * BLK, BLK)], axis=0)[:, None] + p_self
    o = num / den + bias_ref[...]
    nrm = jnp.sqrt(jnp.sum(o * o, axis=1, keepdims=True))
    out_ref[...] = o / jnp.maximum(nrm, 1e-12)


def _tc2(acc, den, h, a2, m_sum, bias):
    return pl.pallas_call(
        _tc2_body,
        grid=(GRID,),
        in_specs=[
            pl.BlockSpec((NUM_SC, BLK, C), lambda i: (0, i, 0)),
            pl.BlockSpec((N_WORKERS, NP), lambda i: (0, 0)),
            pl.BlockSpec((BLK, C), lambda i: (i, 0)),
            pl.BlockSpec((NP, 2), lambda i: (0, 0)),
            pl.BlockSpec((1, C), lambda i: (0, 0)),
            pl.BlockSpec((1, C), lambda i: (0, 0)),
        ],
        out_specs=pl.BlockSpec((BLK, C), lambda i: (i, 0)),
        out_shape=jax.ShapeDtypeStruct((NP, C), jnp.float32),
    )(acc, den, h, a2, m_sum, bias)


@jax.jit
def kernel(x, edge_indices, weight, att, bias):
    attc = jnp.stack([att[0, 0, :C], att[0, 0, C:]], axis=1)
    xp = jnp.pad(x, ((0, NP - N), (0, 0)))
    h, a2, md, ms = _tc1(xp, weight, attc)
    m_sum = md + ms
    src = edge_indices[0].astype(jnp.int32)
    dst = edge_indices[1].astype(jnp.int32)
    adst = a2[:, 0]
    asrc = a2[:, 1]
    m16 = m_sum[0, :16]
    p, den = _sc_attention(adst, asrc, src, dst, m16)
    acc = _sc_message(h, src, dst, p)
    return _tc2(acc, den, h, a2, m_sum, bias.reshape(1, C))[:N]
